# Initial kernel scaffold; baseline (speedup 1.0000x reference)
import jax
import jax.numpy as jnp
from jax.experimental import pallas as pl


def kernel(x, x_chars, classifier):
    raise NotImplementedError("write your pallas kernel here")


# speedup vs baseline: 57.3617x; 57.3617x over previous
import functools

import jax
import jax.numpy as jnp
from jax import lax
from jax.experimental import pallas as pl
from jax.experimental.pallas import tpu as pltpu
from jax.experimental.pallas import tpu_sc as plsc

V = 1000
T = 20
VP = 1024
L = 16

_info = plsc.get_sparse_core_info()
_NC, _NS = _info.num_cores, _info.num_subcores
NW = _NC * _NS


def _sc_body(per_w, ct_hbm, x_hbm, out_hbm, ct_v, tbl_v, idx_v, res_v,
             sem_i, sem_c):
    wid = lax.axis_index("s") * _NC + lax.axis_index("c")
    base = wid * per_w

    h_idx = pltpu.async_copy(x_hbm.at[pl.ds(base, per_w)], idx_v, sem_i)
    h_ct = pltpu.async_copy(ct_hbm, ct_v, sem_c)
    h_ct.wait()

    def chunk(c, carry):
        off = c * L
        best_v = ct_v[0, pl.ds(off, L)]
        best_i = jnp.zeros((L,), jnp.int32)
        for t in range(1, T):
            vals = ct_v[t, pl.ds(off, L)]
            m = vals > best_v
            best_v = jnp.where(m, vals, best_v)
            best_i = jnp.where(m, jnp.full((L,), t, jnp.int32), best_i)
        tbl_v[pl.ds(off, L)] = best_i
        return carry

    lax.fori_loop(0, VP // L, chunk, 0)

    h_idx.wait()

    UN = 8
    def gather_blk(j, carry):
        b0 = j * (L * UN)
        for u in range(UN):
            off = b0 + u * L
            idxs = idx_v[pl.ds(off, L)]
            res_v[pl.ds(off, L)] = plsc.load_gather(tbl_v, [idxs])
        return carry

    lax.fori_loop(0, per_w // (L * UN), gather_blk, 0)

    pltpu.sync_copy(res_v, out_hbm.at[pl.ds(base, per_w)])


def kernel(x, x_chars, classifier):
    del x_chars
    n = x.size
    per_w = n // NW
    ct = jnp.pad(classifier.T, ((0, 0), (0, VP - V)))
    xf = x.reshape(-1)

    k = functools.partial(
        pl.kernel,
        out_type=jax.ShapeDtypeStruct((n,), jnp.int32),
        mesh=plsc.VectorSubcoreMesh(core_axis_name="c", subcore_axis_name="s"),
        compiler_params=pltpu.CompilerParams(needs_layout_passes=False),
        scratch_types=[
            pltpu.VMEM((T, VP), jnp.float32),
            pltpu.VMEM((VP,), jnp.int32),
            pltpu.VMEM((per_w,), jnp.int32),
            pltpu.VMEM((per_w,), jnp.int32),
            pltpu.SemaphoreType.DMA,
            pltpu.SemaphoreType.DMA,
        ],
    )(functools.partial(_sc_body, per_w))

    return k(ct, xf).reshape(x.shape)


# speedup vs baseline: 68.2769x; 1.1903x over previous
import functools

import jax
import jax.numpy as jnp
from jax import lax
from jax.experimental import pallas as pl
from jax.experimental.pallas import tpu as pltpu
from jax.experimental.pallas import tpu_sc as plsc

V = 1000
T = 20
VP = 1024
L = 16
CFP = VP * T

_info = plsc.get_sparse_core_info()
_NC, _NS = _info.num_cores, _info.num_subcores
NW = _NC * _NS


def _sc_body(rows_per_w, seq, cf_hbm, x_hbm, out_hbm, cf_v, tbl_v, idx_v,
             res_v, sem_i, sem_c):
    wid = lax.axis_index("s") * _NC + lax.axis_index("c")
    r0 = wid * rows_per_w

    h_idx = pltpu.async_copy(x_hbm.at[pl.ds(r0, rows_per_w)], idx_v, sem_i)
    h_ct = pltpu.async_copy(cf_hbm, cf_v, sem_c)
    h_ct.wait()

    lanes = lax.iota(jnp.int32, L)

    def chunk(c, carry):
        base = c * (L * T)
        col_idx = lanes * T + base
        best_v = plsc.load_gather(cf_v, [col_idx])
        best_i = jnp.zeros((L,), jnp.int32)
        for t in range(1, T):
            vals = plsc.load_gather(cf_v, [col_idx + t])
            m = vals > best_v
            best_v = jnp.where(m, vals, best_v)
            best_i = jnp.where(m, jnp.full((L,), t, jnp.int32), best_i)
        tbl_v[pl.ds(c * L, L)] = best_i
        return carry

    lax.fori_loop(0, VP // L, chunk, 0)

    h_idx.wait()

    starts = list(range(0, seq - L + 1, L))
    if starts[-1] != seq - L:
        starts.append(seq - L)

    def row_blk(r, carry):
        for c in starts:
            idxs = idx_v[r, pl.ds(c, L)]
            res_v[r, pl.ds(c, L)] = plsc.load_gather(tbl_v, [idxs])
        return carry

    lax.fori_loop(0, rows_per_w, row_blk, 0)

    pltpu.sync_copy(res_v, out_hbm.at[pl.ds(r0, rows_per_w)])


def kernel(x, x_chars, classifier):
    del x_chars
    batch, seq = x.shape
    rows_per_w = batch // NW
    cf = jnp.pad(classifier.reshape(-1), (0, CFP - V * T))

    k = functools.partial(
        pl.kernel,
        out_type=jax.ShapeDtypeStruct((batch, seq), jnp.int32),
        mesh=plsc.VectorSubcoreMesh(core_axis_name="c", subcore_axis_name="s"),
        compiler_params=pltpu.CompilerParams(
            needs_layout_passes=False, use_tc_tiling_on_sc=True),
        scratch_types=[
            pltpu.VMEM((CFP,), jnp.float32),
            pltpu.VMEM((VP,), jnp.int32),
            pltpu.VMEM((rows_per_w, seq), jnp.int32),
            pltpu.VMEM((rows_per_w, seq), jnp.int32),
            pltpu.SemaphoreType.DMA,
            pltpu.SemaphoreType.DMA,
        ],
    )(functools.partial(_sc_body, rows_per_w, seq))

    return k(cf, x)


# speedup vs baseline: 87.6466x; 1.2837x over previous
import functools

import jax
import jax.numpy as jnp
from jax import lax
from jax.experimental import pallas as pl
from jax.experimental.pallas import tpu as pltpu
from jax.experimental.pallas import tpu_sc as plsc

V = 1000
T = 20
VP = 1024
L = 16
CFP = VP * T
VPT = VP // 16

_info = plsc.get_sparse_core_info()
_NC, _NS = _info.num_cores, _info.num_subcores
NW = _NC * _NS


def _sc_body(rows_per_w, seq, cf_hbm, x_hbm, out_hbm, cf_v, tbl_v, idx_v,
             res_v, shr_tbl, sem_i, sem_c, sem_o):
    sub = lax.axis_index("s")
    wid = sub * _NC + lax.axis_index("c")
    r0 = wid * rows_per_w

    h_idx = pltpu.async_copy(x_hbm.at[pl.ds(r0, rows_per_w)], idx_v, sem_i)
    pltpu.async_copy(cf_hbm.at[pl.ds(sub * (VPT * T), VPT * T)], cf_v,
                     sem_c).wait()

    lanes = lax.iota(jnp.int32, L)

    @plsc.parallel_loop(0, VPT // L)
    def _chunk(j):
        col_idx = (lanes + j * L) * T
        best_v = plsc.load_gather(cf_v, [col_idx])
        best_i = jnp.zeros((L,), jnp.int32)
        for t in range(1, T):
            vals = plsc.load_gather(cf_v, [col_idx + t])
            m = vals > best_v
            best_v = jnp.where(m, vals, best_v)
            best_i = jnp.where(m, jnp.full((L,), t, jnp.int32), best_i)
        tbl_v[pl.ds(sub * VPT + j * L, L)] = best_i

    pltpu.sync_copy(tbl_v.at[pl.ds(sub * VPT, VPT)],
                    shr_tbl.at[pl.ds(sub * VPT, VPT)])
    plsc.subcore_barrier()
    pltpu.sync_copy(shr_tbl, tbl_v)

    h_idx.wait()

    starts = list(range(0, seq - L + 1, L))
    if starts[-1] != seq - L:
        starts.append(seq - L)

    groups = 4
    gsz = rows_per_w // groups
    handles = []
    for g in range(groups):

        @plsc.parallel_loop(g * gsz, (g + 1) * gsz)
        def _row(r):
            for c in starts:
                idxs = idx_v[r, pl.ds(c, L)]
                res_v[r, pl.ds(c, L)] = plsc.load_gather(tbl_v, [idxs])

        handles.append(
            pltpu.async_copy(res_v.at[pl.ds(g * gsz, gsz)],
                             out_hbm.at[pl.ds(r0 + g * gsz, gsz)], sem_o))
    for h in handles:
        h.wait()


def kernel(x, x_chars, classifier):
    del x_chars
    batch, seq = x.shape
    rows_per_w = batch // NW
    cf = jnp.pad(classifier.reshape(-1), (0, CFP - V * T))

    k = functools.partial(
        pl.kernel,
        out_type=jax.ShapeDtypeStruct((batch, seq), jnp.int32),
        mesh=plsc.VectorSubcoreMesh(core_axis_name="c", subcore_axis_name="s"),
        compiler_params=pltpu.CompilerParams(
            needs_layout_passes=False, use_tc_tiling_on_sc=True),
        scratch_types=[
            pltpu.VMEM((VPT * T,), jnp.float32),
            pltpu.VMEM((VP,), jnp.int32),
            pltpu.VMEM((rows_per_w, seq), jnp.int32),
            pltpu.VMEM((rows_per_w, seq), jnp.int32),
            pltpu.VMEM_SHARED((VP,), jnp.int32),
            pltpu.SemaphoreType.DMA,
            pltpu.SemaphoreType.DMA,
            pltpu.SemaphoreType.DMA,
        ],
    )(functools.partial(_sc_body, rows_per_w, seq))

    return k(cf, x)


# speedup vs baseline: 126.2242x; 1.4401x over previous
import functools

import jax
import jax.numpy as jnp
from jax import lax
from jax.experimental import pallas as pl
from jax.experimental.pallas import tpu as pltpu
from jax.experimental.pallas import tpu_sc as plsc

V = 1000
T = 20
VP = 1024
L = 16
CFP = VP * T
VPT = VP // 16

_info = plsc.get_sparse_core_info()
_NC, _NS = _info.num_cores, _info.num_subcores
NW = _NC * _NS


def _sc_body(seq, cols_per_w, cf_hbm, xt_hbm, out_hbm, cf_v, tbl_v, idx_v,
             res_v, shr_tbl, sem_i, sem_c, sem_o):
    sub = lax.axis_index("s")
    wid = sub * _NC + lax.axis_index("c")
    c0 = wid * cols_per_w

    h_idx = pltpu.async_copy(xt_hbm.at[:, pl.ds(c0, cols_per_w)], idx_v,
                             sem_i)
    pltpu.async_copy(cf_hbm.at[pl.ds(sub * (VPT * T), VPT * T)], cf_v,
                     sem_c).wait()

    lanes = lax.iota(jnp.int32, L)

    @plsc.parallel_loop(0, VPT // L)
    def _chunk(j):
        col_idx = (lanes + j * L) * T
        best_v = plsc.load_gather(cf_v, [col_idx])
        best_i = jnp.zeros((L,), jnp.int32)
        for t in range(1, T):
            vals = plsc.load_gather(cf_v, [col_idx + t])
            m = vals > best_v
            best_v = jnp.where(m, vals, best_v)
            best_i = jnp.where(m, jnp.full((L,), t, jnp.int32), best_i)
        tbl_v[pl.ds(sub * VPT + j * L, L)] = best_i

    pltpu.sync_copy(tbl_v.at[pl.ds(sub * VPT, VPT)],
                    shr_tbl.at[pl.ds(sub * VPT, VPT)])
    plsc.subcore_barrier()
    pltpu.sync_copy(shr_tbl, tbl_v)

    h_idx.wait()

    vecs = cols_per_w // L
    row_groups = [0, 48, 96, 144, seq]
    handles = []
    for g in range(len(row_groups) - 1):
        lo, hi = row_groups[g], row_groups[g + 1]

        @plsc.parallel_loop(lo, hi)
        def _row(r):
            for u in range(vecs):
                idxs = idx_v[r, pl.ds(u * L, L)]
                res_v[r, pl.ds(u * L, L)] = plsc.load_gather(tbl_v, [idxs])

        handles.append(
            pltpu.async_copy(res_v.at[pl.ds(lo, hi - lo)],
                             out_hbm.at[pl.ds(lo, hi - lo),
                                        pl.ds(c0, cols_per_w)], sem_o))
    for h in handles:
        h.wait()


def kernel(x, x_chars, classifier):
    del x_chars
    batch, seq = x.shape
    cols_per_w = batch // NW
    cf = jnp.pad(classifier.reshape(-1), (0, CFP - V * T))

    k = functools.partial(
        pl.kernel,
        out_type=jax.ShapeDtypeStruct((seq, batch), jnp.int32),
        mesh=plsc.VectorSubcoreMesh(core_axis_name="c", subcore_axis_name="s"),
        compiler_params=pltpu.CompilerParams(
            needs_layout_passes=False, use_tc_tiling_on_sc=True),
        scratch_types=[
            pltpu.VMEM((VPT * T,), jnp.float32),
            pltpu.VMEM((VP,), jnp.int32),
            pltpu.VMEM((seq, cols_per_w), jnp.int32),
            pltpu.VMEM((seq, cols_per_w), jnp.int32),
            pltpu.VMEM_SHARED((VP,), jnp.int32),
            pltpu.SemaphoreType.DMA,
            pltpu.SemaphoreType.DMA,
            pltpu.SemaphoreType.DMA,
        ],
    )(functools.partial(_sc_body, seq, cols_per_w))

    return k(cf, x.T).T
